# baseline (speedup 1.0000x reference)
import functools

import jax
import jax.numpy as jnp
from jax import lax
from jax.experimental import pallas as pl
from jax.experimental.pallas import tpu as pltpu
from jax.experimental.pallas import tpu_sc as plsc

B = 16384
D = 64
W = 128
NC = 2
NS = 16
NW = NC * NS
BPW = B // NW
CHUNK = 128
NCHUNK = BPW // CHUNK


def _neg_sqrt(x):
    i = lax.bitcast_convert_type(x, jnp.int32)
    y = lax.bitcast_convert_type(jnp.int32(0x5F3759DF) - (i >> 1), jnp.float32)
    for _ in range(3):
        y = y * (1.5 - 0.5 * x * y * y)
    return -(x * y)


def _sc_body(hrow_h, rrow_h, trow_h, hoff_h, roff_h, toff_h,
             etab_h, rtab_h, out_h,
             hidx_v, ridx_v, tidx_v, hoff_v, roff_v, toff_v,
             h_v, r_v, t_v, s_v, sem):
    wid = lax.axis_index("s") * NC + lax.axis_index("c")
    base = wid * BPW
    pltpu.sync_copy(hrow_h.at[pl.ds(wid * NCHUNK, NCHUNK)], hidx_v)
    pltpu.sync_copy(rrow_h.at[pl.ds(wid * NCHUNK, NCHUNK)], ridx_v)
    pltpu.sync_copy(trow_h.at[pl.ds(wid * NCHUNK, NCHUNK)], tidx_v)
    pltpu.sync_copy(hoff_h.at[pl.ds(base, BPW)], hoff_v)
    pltpu.sync_copy(roff_h.at[pl.ds(base, BPW)], roff_v)
    pltpu.sync_copy(toff_h.at[pl.ds(base, BPW)], toff_v)

    lane = lax.iota(jnp.int32, 16)

    def chunk_compute(j, carry):
        def group(g, carry):
            rows = g * 16 + lane
            ho = hoff_v[pl.ds(j * CHUNK + g * 16, 16)]
            ro = roff_v[pl.ds(j * CHUNK + g * 16, 16)]
            to = toff_v[pl.ds(j * CHUNK + g * 16, 16)]
            acc = jnp.zeros((16,), jnp.float32)
            for d in range(D):
                hv = plsc.load_gather(h_v, [rows, ho + d])
                rv = plsc.load_gather(r_v, [rows, ro + d])
                tv = plsc.load_gather(t_v, [rows, to + d])
                dd = hv + rv - tv
                acc = acc + dd * dd
            s_v[pl.ds(j * CHUNK + g * 16, 16)] = _neg_sqrt(acc)
            return carry
        lax.fori_loop(0, CHUNK // 16, group, 0)
        return carry

    for j in range(NCHUNK):
        pltpu.async_copy(etab_h.at[hidx_v.at[j]], h_v, sem)
        pltpu.async_copy(rtab_h.at[ridx_v.at[j]], r_v, sem)
        pltpu.async_copy(etab_h.at[tidx_v.at[j]], t_v, sem)
        pltpu.make_async_copy(etab_h.at[hidx_v.at[j]], h_v, sem).wait()
        pltpu.make_async_copy(rtab_h.at[ridx_v.at[j]], r_v, sem).wait()
        pltpu.make_async_copy(etab_h.at[tidx_v.at[j]], t_v, sem).wait()
        chunk_compute(j, 0)

    pltpu.sync_copy(s_v, out_h.at[pl.ds(base, BPW)])


@jax.jit
def _sc_call(hrow, rrow, trow, hoff, roff, toff, etab2, rtab2):
    mesh = plsc.VectorSubcoreMesh(core_axis_name="c", subcore_axis_name="s")
    run = functools.partial(
        pl.kernel,
        out_type=jax.ShapeDtypeStruct((B,), jnp.float32),
        mesh=mesh,
        compiler_params=pltpu.CompilerParams(needs_layout_passes=False),
        scratch_types=[
            pltpu.VMEM((NCHUNK, CHUNK), jnp.int32),
            pltpu.VMEM((NCHUNK, CHUNK), jnp.int32),
            pltpu.VMEM((NCHUNK, CHUNK), jnp.int32),
            pltpu.VMEM((BPW,), jnp.int32),
            pltpu.VMEM((BPW,), jnp.int32),
            pltpu.VMEM((BPW,), jnp.int32),
            pltpu.VMEM((CHUNK, W), jnp.float32),
            pltpu.VMEM((CHUNK, W), jnp.float32),
            pltpu.VMEM((CHUNK, W), jnp.float32),
            pltpu.VMEM((BPW,), jnp.float32),
            pltpu.SemaphoreType.DMA,
        ],
    )(_sc_body)
    return run(hrow, rrow, trow, hoff, roff, toff, etab2, rtab2)


def kernel(heads, relations, tails, entity_table, relation_table):
    hrow = (heads >> 1).reshape(B // CHUNK, CHUNK)
    rrow = (relations >> 1).reshape(B // CHUNK, CHUNK)
    trow = (tails >> 1).reshape(B // CHUNK, CHUNK)
    hoff = (heads & 1) * D
    roff = (relations & 1) * D
    toff = (tails & 1) * D
    etab2 = entity_table.reshape(-1, W)
    rtab2 = relation_table.reshape(-1, W)
    return _sc_call(hrow, rrow, trow, hoff, roff, toff, etab2, rtab2)
